# baseline (speedup 1.0000x reference)
import jax
import jax.numpy as jnp
from jax import lax
from jax.experimental import pallas as pl
from jax.experimental.pallas import tpu as pltpu
from jax.experimental.pallas import tpu_sc as plsc

N = 10000
E = 320000
D = 128
G = 64
DOUT = 64

NC = 2
NS = 16
NW = NC * NS
EPW = E // NW
K = 80
CH = EPW // K
NSEG = 5
CHS = CH // NSEG
NB = 4
RCH = 80
NRCH = N // RCH
RROUNDS = -(-NRCH // NS)

BLK = 1000


DWIN = 8


def _deg_body(dst_hbm, const_hbm, out_hbm, dst_v, ones_v, deg_sh, semz, semd):
    c = lax.axis_index("c")
    s = lax.axis_index("s")
    wid = s * NC + c

    for r in range(RROUNDS):
        cid = r * NS + s

        @pl.when(cid < NRCH)
        def _():
            pltpu.async_copy(const_hbm.at[pl.ds(0, RCH)],
                             deg_sh.at[pl.ds(cid * RCH, RCH)], semz)

    pltpu.async_copy(const_hbm.at[pl.ds(RCH, K)], ones_v, semz)
    pltpu.async_copy(dst_hbm.at[wid], dst_v, semz)
    for r in range(RROUNDS):
        cid = r * NS + s

        @pl.when(cid < NRCH)
        def _():
            pltpu.make_async_copy(const_hbm.at[pl.ds(0, RCH)],
                                  deg_sh.at[pl.ds(cid * RCH, RCH)], semz).wait()
    pltpu.make_async_copy(const_hbm.at[pl.ds(RCH, K)], ones_v, semz).wait()
    pltpu.make_async_copy(dst_hbm.at[wid], dst_v, semz).wait()
    plsc.subcore_barrier()

    def body(j, carry):
        @pl.when(j >= DWIN)
        def _():
            pltpu.make_async_copy(ones_v, deg_sh.at[dst_v.at[0]], semd).wait()

        pltpu.async_copy(ones_v, deg_sh.at[dst_v.at[j]], semd, add=True)
        return carry

    lax.fori_loop(0, CH, body, 0)
    for _ in range(DWIN):
        pltpu.make_async_copy(ones_v, deg_sh.at[dst_v.at[0]], semd).wait()
    plsc.subcore_barrier()

    for r in range(RROUNDS):
        cid = r * NS + s

        @pl.when(cid < NRCH)
        def _():
            off = cid * RCH
            pltpu.sync_copy(deg_sh.at[pl.ds(off, RCH)],
                            out_hbm.at[c, pl.ds(off, RCH)])


def _deg_partials(dst2d, deg_const):
    return pl.kernel(
        _deg_body,
        out_type=jax.ShapeDtypeStruct((NC, N, D), jnp.float32),
        mesh=plsc.VectorSubcoreMesh(core_axis_name="c", subcore_axis_name="s"),
        scratch_types=[
            pltpu.VMEM((CH, K), jnp.int32),
            pltpu.VMEM((K, D), jnp.float32),
            pltpu.VMEM_SHARED((N, D), jnp.float32),
            pltpu.SemaphoreType.DMA,
            pltpu.SemaphoreType.DMA,
        ],
    )(dst2d, deg_const)


def _prep_body(x_ref, w_ref, degp_ref, y_ref, dinv_ref):
    degp = degp_ref[...]
    deg = degp[0, :, 0] + degp[1, :, 0] + 1.0
    dinv = lax.rsqrt(deg)
    xw = jnp.dot(x_ref[...], w_ref[...], preferred_element_type=jnp.float32)
    y_ref[...] = xw * dinv[:, None]
    dinv_ref[...] = dinv[:, None]


def _prep(x, w_conv, deg_part):
    return pl.pallas_call(
        _prep_body,
        grid=(N // BLK,),
        in_specs=[
            pl.BlockSpec((BLK, D), lambda i: (i, 0)),
            pl.BlockSpec((D, D), lambda i: (0, 0)),
            pl.BlockSpec((NC, BLK, D), lambda i: (0, i, 0)),
        ],
        out_specs=[
            pl.BlockSpec((BLK, D), lambda i: (i, 0)),
            pl.BlockSpec((BLK, 1), lambda i: (i, 0)),
        ],
        out_shape=[
            jax.ShapeDtypeStruct((N, D), jnp.float32),
            jax.ShapeDtypeStruct((N, 1), jnp.float32),
        ],
    )(x, w_conv, deg_part)


def _edge_body(y_hbm, src_hbm, dst_hbm, zer_hbm, out_hbm, src_v, dst_v, rows_v,
               acc_sh, semz, semg, sems):
    c = lax.axis_index("c")
    s = lax.axis_index("s")
    wid = s * NC + c

    for r in range(RROUNDS):
        cid = r * NS + s

        @pl.when(cid < NRCH)
        def _():
            pltpu.async_copy(zer_hbm, acc_sh.at[pl.ds(cid * RCH, RCH)], semz)

    for r in range(RROUNDS):
        cid = r * NS + s

        @pl.when(cid < NRCH)
        def _():
            pltpu.make_async_copy(zer_hbm, acc_sh.at[pl.ds(cid * RCH, RCH)],
                                  semz).wait()
    plsc.subcore_barrier()

    def gather(j, b):
        return pltpu.async_copy(y_hbm.at[src_v.at[j]], rows_v.at[b], semg.at[b])

    def gather_wait(j, b):
        pltpu.make_async_copy(y_hbm.at[src_v.at[j]], rows_v.at[b],
                              semg.at[b]).wait()

    def scat(j, b):
        return pltpu.async_copy(rows_v.at[b], acc_sh.at[dst_v.at[j]],
                                sems.at[b], add=True)

    def scat_wait(j, b):
        pltpu.make_async_copy(rows_v.at[b], acc_sh.at[dst_v.at[j]],
                              sems.at[b]).wait()

    for h in range(NSEG):
        pltpu.async_copy(src_hbm.at[wid, h], src_v, semz)
        pltpu.async_copy(dst_hbm.at[wid, h], dst_v, semz)
        pltpu.make_async_copy(src_hbm.at[wid, h], src_v, semz).wait()
        pltpu.make_async_copy(dst_hbm.at[wid, h], dst_v, semz).wait()

        for m in range(NB - 1):
            gather(m, m)

        def body(p, carry):
            j = p * NB
            for t in range(NB):
                b = t

                @pl.when(j + t < CHS)
                def _():
                    gather_wait(j + t, b)
                    scat(j + t, b)

                    @pl.when(j + t + NB - 1 < CHS)
                    def _():
                        @pl.when(j + t - 1 >= 0)
                        def _():
                            scat_wait(j + t - 1, (t + NB - 1) % NB)

                        gather(j + t + NB - 1, (t + NB - 1) % NB)

            return carry

        lax.fori_loop(0, -(-CHS // NB), body, 0)
        for t in range(NB):
            scat_wait(CHS - 1 - t, (CHS - 1 - t) % NB)
    plsc.subcore_barrier()

    for r in range(RROUNDS):
        cid = r * NS + s

        @pl.when(cid < NRCH)
        def _():
            off = cid * RCH
            pltpu.sync_copy(acc_sh.at[pl.ds(off, RCH)],
                            out_hbm.at[c, pl.ds(off, RCH)])


def _edge_scatter(y, src2d, dst2d, zer_hbm):
    return pl.kernel(
        _edge_body,
        out_type=jax.ShapeDtypeStruct((NC, N, D), jnp.float32),
        mesh=plsc.VectorSubcoreMesh(core_axis_name="c", subcore_axis_name="s"),
        scratch_types=[
            pltpu.VMEM((CHS, K), jnp.int32),
            pltpu.VMEM((CHS, K), jnp.int32),
            pltpu.VMEM((NB, K, D), jnp.float32),
            pltpu.VMEM_SHARED((N, D), jnp.float32),
            pltpu.SemaphoreType.DMA,
            pltpu.SemaphoreType.DMA((NB,)),
            pltpu.SemaphoreType.DMA((NB,)),
        ],
    )(y, src2d, dst2d, zer_hbm)


def _final_body(acc_ref, y_ref, dinv_ref, batch_ref, bconv_ref, wlin_ref,
                blin_ref, out_ref, sums_sc, cnt_sc):
    i = pl.program_id(0)

    @pl.when(i == 0)
    def _():
        sums_sc[...] = jnp.zeros_like(sums_sc)
        cnt_sc[...] = jnp.zeros_like(cnt_sc)

    acc = acc_ref[0] + acc_ref[1] + y_ref[...]
    h = jnp.maximum(acc * dinv_ref[...] + bconv_ref[...], 0.0)
    onehot = (lax.broadcasted_iota(jnp.int32, (G, BLK), 0)
              == batch_ref[...][:, 0][None, :]).astype(jnp.float32)
    sums_sc[...] += jnp.dot(onehot, h, preferred_element_type=jnp.float32)
    cnt_sc[...] += jnp.sum(onehot, axis=1, keepdims=True)

    @pl.when(i == pl.num_programs(0) - 1)
    def _():
        emb = sums_sc[...] / jnp.maximum(cnt_sc[...], 1.0)
        out_ref[...] = jnp.tanh(
            jnp.dot(emb, wlin_ref[...], preferred_element_type=jnp.float32)
            + blin_ref[...])


def _finalize(acc, y, dinv, batch2d, b_conv2d, w_lin, b_lin2d):
    return pl.pallas_call(
        _final_body,
        grid=(N // BLK,),
        in_specs=[
            pl.BlockSpec((NC, BLK, D), lambda i: (0, i, 0)),
            pl.BlockSpec((BLK, D), lambda i: (i, 0)),
            pl.BlockSpec((BLK, 1), lambda i: (i, 0)),
            pl.BlockSpec((BLK, 1), lambda i: (i, 0)),
            pl.BlockSpec((1, D), lambda i: (0, 0)),
            pl.BlockSpec((D, DOUT), lambda i: (0, 0)),
            pl.BlockSpec((1, DOUT), lambda i: (0, 0)),
        ],
        out_specs=pl.BlockSpec((G, DOUT), lambda i: (0, 0)),
        out_shape=jax.ShapeDtypeStruct((G, DOUT), jnp.float32),
        scratch_shapes=[
            pltpu.VMEM((G, D), jnp.float32),
            pltpu.VMEM((G, 1), jnp.float32),
        ],
    )(acc, y, dinv, batch2d, b_conv2d, w_lin, b_lin2d)


def kernel(x, edge_index, batch, W_conv, b_conv, W_lin, b_lin):
    src2d = edge_index[0].astype(jnp.int32).reshape(NW, NSEG, CHS, K)
    dst2d = edge_index[1].astype(jnp.int32).reshape(NW, NSEG, CHS, K)
    deg_const = jnp.concatenate(
        [jnp.zeros((RCH, D), jnp.float32), jnp.ones((K, D), jnp.float32)])
    deg_part = _deg_partials(dst2d.reshape(NW, CH, K), deg_const)
    y, dinv = _prep(x, W_conv, deg_part)
    acc = _edge_scatter(y, src2d, dst2d, jnp.zeros((RCH, D), jnp.float32))
    return _finalize(
        acc, y, dinv,
        batch.astype(jnp.int32).reshape(N, 1),
        b_conv.reshape(1, D), W_lin, b_lin.reshape(1, DOUT),
    )


# speedup vs baseline: 1.0194x; 1.0194x over previous
import jax
import jax.numpy as jnp
from jax import lax
from jax.experimental import pallas as pl
from jax.experimental.pallas import tpu as pltpu
from jax.experimental.pallas import tpu_sc as plsc

N = 10000
E = 320000
D = 128
G = 64
DOUT = 64

NC = 2
NS = 16
NW = NC * NS
EPW = E // NW
K = 100
CH = EPW // K
NSEG = 5
CHS = CH // NSEG
NB = 3
RCH = 80
NRCH = N // RCH
RROUNDS = -(-NRCH // NS)

BLK = 1000


DWIN = 8


def _deg_body(dst_hbm, const_hbm, out_hbm, dst_v, ones_v, deg_sh, semz, semd):
    c = lax.axis_index("c")
    s = lax.axis_index("s")
    wid = s * NC + c

    for r in range(RROUNDS):
        cid = r * NS + s

        @pl.when(cid < NRCH)
        def _():
            pltpu.async_copy(const_hbm.at[pl.ds(0, RCH)],
                             deg_sh.at[pl.ds(cid * RCH, RCH)], semz)

    pltpu.async_copy(const_hbm.at[pl.ds(RCH, K)], ones_v, semz)
    pltpu.async_copy(dst_hbm.at[wid], dst_v, semz)
    for r in range(RROUNDS):
        cid = r * NS + s

        @pl.when(cid < NRCH)
        def _():
            pltpu.make_async_copy(const_hbm.at[pl.ds(0, RCH)],
                                  deg_sh.at[pl.ds(cid * RCH, RCH)], semz).wait()
    pltpu.make_async_copy(const_hbm.at[pl.ds(RCH, K)], ones_v, semz).wait()
    pltpu.make_async_copy(dst_hbm.at[wid], dst_v, semz).wait()
    plsc.subcore_barrier()

    def body(j, carry):
        @pl.when(j >= DWIN)
        def _():
            pltpu.make_async_copy(ones_v, deg_sh.at[dst_v.at[0]], semd).wait()

        pltpu.async_copy(ones_v, deg_sh.at[dst_v.at[j]], semd, add=True)
        return carry

    lax.fori_loop(0, CH, body, 0)
    for _ in range(DWIN):
        pltpu.make_async_copy(ones_v, deg_sh.at[dst_v.at[0]], semd).wait()
    plsc.subcore_barrier()

    for r in range(RROUNDS):
        cid = r * NS + s

        @pl.when(cid < NRCH)
        def _():
            off = cid * RCH
            pltpu.sync_copy(deg_sh.at[pl.ds(off, RCH)],
                            out_hbm.at[c, pl.ds(off, RCH)])


def _deg_partials(dst2d, deg_const):
    return pl.kernel(
        _deg_body,
        out_type=jax.ShapeDtypeStruct((NC, N, D), jnp.float32),
        mesh=plsc.VectorSubcoreMesh(core_axis_name="c", subcore_axis_name="s"),
        scratch_types=[
            pltpu.VMEM((CH, K), jnp.int32),
            pltpu.VMEM((K, D), jnp.float32),
            pltpu.VMEM_SHARED((N, D), jnp.float32),
            pltpu.SemaphoreType.DMA,
            pltpu.SemaphoreType.DMA,
        ],
    )(dst2d, deg_const)


def _prep_body(x_ref, w_ref, degp_ref, y_ref, dinv_ref):
    degp = degp_ref[...]
    deg = degp[0, :, 0] + degp[1, :, 0] + 1.0
    dinv = lax.rsqrt(deg)
    xw = jnp.dot(x_ref[...], w_ref[...], preferred_element_type=jnp.float32)
    y_ref[...] = xw * dinv[:, None]
    dinv_ref[...] = dinv[:, None]


def _prep(x, w_conv, deg_part):
    return pl.pallas_call(
        _prep_body,
        grid=(N // BLK,),
        in_specs=[
            pl.BlockSpec((BLK, D), lambda i: (i, 0)),
            pl.BlockSpec((D, D), lambda i: (0, 0)),
            pl.BlockSpec((NC, BLK, D), lambda i: (0, i, 0)),
        ],
        out_specs=[
            pl.BlockSpec((BLK, D), lambda i: (i, 0)),
            pl.BlockSpec((BLK, 1), lambda i: (i, 0)),
        ],
        out_shape=[
            jax.ShapeDtypeStruct((N, D), jnp.float32),
            jax.ShapeDtypeStruct((N, 1), jnp.float32),
        ],
    )(x, w_conv, deg_part)


def _edge_body(y_hbm, src_hbm, dst_hbm, zer_hbm, out_hbm, src_v, dst_v, rows_v,
               acc_sh, semz, semg, sems):
    c = lax.axis_index("c")
    s = lax.axis_index("s")
    wid = s * NC + c

    for r in range(RROUNDS):
        cid = r * NS + s

        @pl.when(cid < NRCH)
        def _():
            pltpu.async_copy(zer_hbm, acc_sh.at[pl.ds(cid * RCH, RCH)], semz)

    for r in range(RROUNDS):
        cid = r * NS + s

        @pl.when(cid < NRCH)
        def _():
            pltpu.make_async_copy(zer_hbm, acc_sh.at[pl.ds(cid * RCH, RCH)],
                                  semz).wait()
    plsc.subcore_barrier()

    def gather(j, b):
        return pltpu.async_copy(y_hbm.at[src_v.at[j]], rows_v.at[b], semg.at[b])

    def gather_wait(j, b):
        pltpu.make_async_copy(y_hbm.at[src_v.at[j]], rows_v.at[b],
                              semg.at[b]).wait()

    def scat(j, b):
        return pltpu.async_copy(rows_v.at[b], acc_sh.at[dst_v.at[j]],
                                sems.at[b], add=True)

    def scat_wait(j, b):
        pltpu.make_async_copy(rows_v.at[b], acc_sh.at[dst_v.at[j]],
                              sems.at[b]).wait()

    for h in range(NSEG):
        pltpu.async_copy(src_hbm.at[wid, h], src_v, semz)
        pltpu.async_copy(dst_hbm.at[wid, h], dst_v, semz)
        pltpu.make_async_copy(src_hbm.at[wid, h], src_v, semz).wait()
        pltpu.make_async_copy(dst_hbm.at[wid, h], dst_v, semz).wait()

        for m in range(NB - 1):
            gather(m, m)

        def body(p, carry):
            j = p * NB
            for t in range(NB):
                b = t

                @pl.when(j + t < CHS)
                def _():
                    gather_wait(j + t, b)
                    scat(j + t, b)

                    @pl.when(j + t + NB - 1 < CHS)
                    def _():
                        @pl.when(j + t - 1 >= 0)
                        def _():
                            scat_wait(j + t - 1, (t + NB - 1) % NB)

                        gather(j + t + NB - 1, (t + NB - 1) % NB)

            return carry

        lax.fori_loop(0, -(-CHS // NB), body, 0)
        for t in range(NB):
            scat_wait(CHS - 1 - t, (CHS - 1 - t) % NB)
    plsc.subcore_barrier()

    for r in range(RROUNDS):
        cid = r * NS + s

        @pl.when(cid < NRCH)
        def _():
            off = cid * RCH
            pltpu.sync_copy(acc_sh.at[pl.ds(off, RCH)],
                            out_hbm.at[c, pl.ds(off, RCH)])


def _edge_scatter(y, src2d, dst2d, zer_hbm):
    return pl.kernel(
        _edge_body,
        out_type=jax.ShapeDtypeStruct((NC, N, D), jnp.float32),
        mesh=plsc.VectorSubcoreMesh(core_axis_name="c", subcore_axis_name="s"),
        scratch_types=[
            pltpu.VMEM((CHS, K), jnp.int32),
            pltpu.VMEM((CHS, K), jnp.int32),
            pltpu.VMEM((NB, K, D), jnp.float32),
            pltpu.VMEM_SHARED((N, D), jnp.float32),
            pltpu.SemaphoreType.DMA,
            pltpu.SemaphoreType.DMA((NB,)),
            pltpu.SemaphoreType.DMA((NB,)),
        ],
    )(y, src2d, dst2d, zer_hbm)


def _final_body(acc_ref, y_ref, dinv_ref, batch_ref, bconv_ref, wlin_ref,
                blin_ref, out_ref, sums_sc, cnt_sc):
    i = pl.program_id(0)

    @pl.when(i == 0)
    def _():
        sums_sc[...] = jnp.zeros_like(sums_sc)
        cnt_sc[...] = jnp.zeros_like(cnt_sc)

    acc = acc_ref[0] + acc_ref[1] + y_ref[...]
    h = jnp.maximum(acc * dinv_ref[...] + bconv_ref[...], 0.0)
    onehot = (lax.broadcasted_iota(jnp.int32, (G, BLK), 0)
              == batch_ref[...][:, 0][None, :]).astype(jnp.float32)
    sums_sc[...] += jnp.dot(onehot, h, preferred_element_type=jnp.float32)
    cnt_sc[...] += jnp.sum(onehot, axis=1, keepdims=True)

    @pl.when(i == pl.num_programs(0) - 1)
    def _():
        emb = sums_sc[...] / jnp.maximum(cnt_sc[...], 1.0)
        out_ref[...] = jnp.tanh(
            jnp.dot(emb, wlin_ref[...], preferred_element_type=jnp.float32)
            + blin_ref[...])


def _finalize(acc, y, dinv, batch2d, b_conv2d, w_lin, b_lin2d):
    return pl.pallas_call(
        _final_body,
        grid=(N // BLK,),
        in_specs=[
            pl.BlockSpec((NC, BLK, D), lambda i: (0, i, 0)),
            pl.BlockSpec((BLK, D), lambda i: (i, 0)),
            pl.BlockSpec((BLK, 1), lambda i: (i, 0)),
            pl.BlockSpec((BLK, 1), lambda i: (i, 0)),
            pl.BlockSpec((1, D), lambda i: (0, 0)),
            pl.BlockSpec((D, DOUT), lambda i: (0, 0)),
            pl.BlockSpec((1, DOUT), lambda i: (0, 0)),
        ],
        out_specs=pl.BlockSpec((G, DOUT), lambda i: (0, 0)),
        out_shape=jax.ShapeDtypeStruct((G, DOUT), jnp.float32),
        scratch_shapes=[
            pltpu.VMEM((G, D), jnp.float32),
            pltpu.VMEM((G, 1), jnp.float32),
        ],
    )(acc, y, dinv, batch2d, b_conv2d, w_lin, b_lin2d)


def kernel(x, edge_index, batch, W_conv, b_conv, W_lin, b_lin):
    src2d = edge_index[0].astype(jnp.int32).reshape(NW, NSEG, CHS, K)
    dst2d = edge_index[1].astype(jnp.int32).reshape(NW, NSEG, CHS, K)
    deg_const = jnp.concatenate(
        [jnp.zeros((RCH, D), jnp.float32), jnp.ones((K, D), jnp.float32)])
    deg_part = _deg_partials(dst2d.reshape(NW, CH, K), deg_const)
    y, dinv = _prep(x, W_conv, deg_part)
    acc = _edge_scatter(y, src2d, dst2d, jnp.zeros((RCH, D), jnp.float32))
    return _finalize(
        acc, y, dinv,
        batch.astype(jnp.int32).reshape(N, 1),
        b_conv.reshape(1, D), W_lin, b_lin.reshape(1, DOUT),
    )
